# baseline (speedup 1.0000x reference)
import jax
import jax.numpy as jnp
from jax import lax
from jax.experimental import pallas as pl
from jax.experimental.pallas import tpu as pltpu
from jax.experimental.pallas import tpu_sc as plsc

N = 10000
E = 160000
IND = 256
HID = 512
OUT = 64
H = 4
EPS = 1e-5

ROW_BLK = 10000
NB = N // ROW_BLK

CHUNK = 128
NSUB = 16
NCHUNK = -(-E // (CHUNK * NSUB)) * NSUB
E_PAD = NCHUNK * CHUNK
CPT = NCHUNK // NSUB
R_AGG = 10112


def _gram_affine_body(h_ref, w1_ref, g1_ref, b1_ref, a_ref, c_ref,
                      g_acc, s_acc):
    @pl.when(pl.program_id(0) == 0)
    def _():
        g_acc[...] = jnp.zeros_like(g_acc)
        s_acc[...] = jnp.zeros_like(s_acc)

    hb = h_ref[...]
    g_acc[...] += lax.dot_general(hb, hb, (((0,), (0,)), ((), ())),
                                  preferred_element_type=jnp.float32)
    s_acc[...] += jnp.sum(hb, axis=0, keepdims=True)

    @pl.when(pl.program_id(0) == NB - 1)
    def _():
        g = g_acc[...]
        s = s_acc[...]
        for i in range(H):
            w = w1_ref[i]
            gw = jnp.dot(g, w, preferred_element_type=jnp.float32)
            m2 = jnp.sum(w * gw, axis=0, keepdims=True) / N
            mu = jnp.dot(s, w, preferred_element_type=jnp.float32) / N
            rstd = lax.rsqrt(m2 - mu * mu + EPS)
            a = g1_ref[i] * rstd
            a_ref[i] = a
            c_ref[i] = b1_ref[i] - mu * a


def _mlp_body(h_ref, w1_ref, a_ref, c_ref, w2_ref, z_ref):
    hb = h_ref[...].astype(jnp.bfloat16)
    for i in range(H):
        t = jnp.dot(hb, w1_ref[i].astype(jnp.bfloat16),
                    preferred_element_type=jnp.float32)
        y = jnp.maximum(t * a_ref[i] + c_ref[i], 0.0)
        z_ref[:, i * OUT:(i + 1) * OUT] = jnp.dot(
            y.astype(jnp.bfloat16), w2_ref[i].astype(jnp.bfloat16),
            preferred_element_type=jnp.float32)


def _aggstat_body(a_ref, s1_ref, s2_ref):
    @pl.when(pl.program_id(0) == 0)
    def _():
        s1_ref[...] = jnp.zeros_like(s1_ref)
        s2_ref[...] = jnp.zeros_like(s2_ref)

    ab = a_ref[...]
    s1_ref[...] += jnp.sum(ab, axis=1, keepdims=True)
    s2_ref[...] += jnp.sum(ab * ab, axis=1, keepdims=True)


def _final_body(a_ref, s1_ref, s2_ref, g_ref, b_ref, h_ref, o_ref):
    mu = s1_ref[...] / N
    rstd = lax.rsqrt(s2_ref[...] / N - mu * mu + EPS)
    y = jnp.maximum((a_ref[...] - mu) * (rstd * g_ref[...]) + b_ref[...], 0.0)
    o_ref[...] = h_ref[...] + jnp.concatenate([y[0], y[1]], axis=1)


def _segsum_body(zt_hbm, idx_hbm, zeros_hbm, out_hbm,
                 idx_v, rows, agg_sh, gsem, ssem, isem):
    cid = lax.axis_index("c")
    sid = lax.axis_index("s")
    base = cid * NCHUNK + sid * CPT

    def start_idx(j, b):
        pltpu.async_copy(idx_hbm.at[base + j], idx_v[b], isem[b])

    def wait_idx(j, b):
        pltpu.make_async_copy(idx_hbm.at[base + j], idx_v[b], isem[b]).wait()

    def start_gather(b, r):
        pltpu.async_copy(zt_hbm.at[idx_v[b].at[0]], rows[r], gsem)

    def wait_gather(b, r):
        pltpu.make_async_copy(zt_hbm.at[idx_v[b].at[0]], rows[r],
                              gsem).wait()

    def start_scatter(b, r):
        pltpu.async_copy(rows[r], agg_sh.at[idx_v[b].at[1]], ssem, add=True)

    def wait_scatter(b, r):
        pltpu.make_async_copy(rows[r], agg_sh.at[idx_v[b].at[1]],
                              ssem).wait()

    zrows = R_AGG // NSUB
    for b in range(4):
        start_idx(b, b)
    pltpu.sync_copy(zeros_hbm.at[pl.ds(sid * zrows, zrows)],
                    agg_sh.at[pl.ds(sid * zrows, zrows)])
    wait_idx(0, 0)
    start_gather(0, 0)
    plsc.subcore_barrier()

    def half(j, b):
        r = b % 2
        wait_gather(b, r)

        @pl.when(j > 0)
        def _():
            wait_scatter((b + 3) % 4, 1 - r)

            @pl.when(j + 3 < CPT)
            def _():
                start_idx(j + 3, (b + 3) % 4)

        start_scatter(b, r)

        @pl.when(j + 1 < CPT)
        def _():
            wait_idx(j + 1, (b + 1) % 4)
            start_gather((b + 1) % 4, 1 - r)

    def body(k, carry):
        for b in range(4):
            j = 4 * k + b

            @pl.when(j < CPT)
            def _():
                half(j, b)

        return carry

    lax.fori_loop(0, (CPT + 3) // 4, body, 0)
    wait_scatter((CPT - 1) % 4, (CPT - 1) % 2)
    plsc.subcore_barrier()

    pltpu.sync_copy(agg_sh.at[pl.ds(sid * zrows, zrows)],
                    out_hbm.at[cid, pl.ds(sid * zrows, zrows)])


@jax.jit
def kernel(h, e, edge_index, W1, g1, b1, W2, gh, bh):
    f32 = jnp.float32

    a1, c1 = pl.pallas_call(
        _gram_affine_body,
        grid=(NB,),
        in_specs=[pl.BlockSpec((ROW_BLK, IND), lambda i: (i, 0)),
                  pl.BlockSpec((H, IND, HID), lambda i: (0, 0, 0)),
                  pl.BlockSpec((H, 1, HID), lambda i: (0, 0, 0)),
                  pl.BlockSpec((H, 1, HID), lambda i: (0, 0, 0))],
        out_specs=[pl.BlockSpec((H, 1, HID), lambda i: (0, 0, 0)),
                   pl.BlockSpec((H, 1, HID), lambda i: (0, 0, 0))],
        out_shape=[jax.ShapeDtypeStruct((H, 1, HID), f32),
                   jax.ShapeDtypeStruct((H, 1, HID), f32)],
        scratch_shapes=[pltpu.VMEM((IND, IND), f32),
                        pltpu.VMEM((1, IND), f32)],
    )(h, W1, g1.reshape(H, 1, HID), b1.reshape(H, 1, HID))

    z_all = pl.pallas_call(
        _mlp_body,
        grid=(NB,),
        in_specs=[pl.BlockSpec((ROW_BLK, IND), lambda i: (i, 0)),
                  pl.BlockSpec((H, IND, HID), lambda i: (0, 0, 0)),
                  pl.BlockSpec((H, 1, HID), lambda i: (0, 0, 0)),
                  pl.BlockSpec((H, 1, HID), lambda i: (0, 0, 0)),
                  pl.BlockSpec((H, HID, OUT), lambda i: (0, 0, 0))],
        out_specs=pl.BlockSpec((ROW_BLK, H * OUT), lambda i: (i, 0)),
        out_shape=jax.ShapeDtypeStruct((N, H * OUT), f32),
    )(h, W1, a1, c1, W2)

    src = edge_index[0]
    dst = edge_index[1]
    pad = E_PAD - E
    srcp = jnp.concatenate([src, jnp.zeros((pad,), jnp.int32)])
    dstp = jnp.concatenate(
        [dst, jnp.full((pad,), R_AGG - 1, jnp.int32)]).reshape(NCHUNK, CHUNK)
    g0 = (2 * srcp).reshape(NCHUNK, CHUNK)
    idx = jnp.stack([jnp.stack([g0, dstp], axis=1),
                     jnp.stack([g0 + 1, dstp], axis=1)])
    idx = idx.reshape(2 * NCHUNK, 2, CHUNK)
    zt = z_all.reshape(2 * N, 128)
    zeros = jnp.zeros((R_AGG, 128), f32)

    mesh = plsc.VectorSubcoreMesh(core_axis_name="c", subcore_axis_name="s")
    agg2 = pl.kernel(
        _segsum_body,
        out_type=jax.ShapeDtypeStruct((2, R_AGG, 128), f32),
        mesh=mesh,
        scratch_types=[
            [pltpu.VMEM((2, CHUNK), jnp.int32) for _ in range(4)],
            [pltpu.VMEM((CHUNK, 128), f32) for _ in range(2)],
            pltpu.VMEM_SHARED((R_AGG, 128), f32),
            pltpu.SemaphoreType.DMA,
            pltpu.SemaphoreType.DMA,
            [pltpu.SemaphoreType.DMA for _ in range(4)],
        ],
    )(zt, idx, zeros)

    s1, s2 = pl.pallas_call(
        _aggstat_body,
        grid=(NB,),
        in_specs=[pl.BlockSpec((2, ROW_BLK, 128), lambda i: (0, i, 0))],
        out_specs=[pl.BlockSpec((2, 1, 128), lambda i: (0, 0, 0)),
                   pl.BlockSpec((2, 1, 128), lambda i: (0, 0, 0))],
        out_shape=[jax.ShapeDtypeStruct((2, 1, 128), f32),
                   jax.ShapeDtypeStruct((2, 1, 128), f32)],
    )(agg2)

    ghr = gh.reshape(2, 1, 128)
    bhr = bh.reshape(2, 1, 128)
    out = pl.pallas_call(
        _final_body,
        grid=(NB,),
        in_specs=[pl.BlockSpec((2, ROW_BLK, 128), lambda i: (0, i, 0)),
                  pl.BlockSpec((2, 1, 128), lambda i: (0, 0, 0)),
                  pl.BlockSpec((2, 1, 128), lambda i: (0, 0, 0)),
                  pl.BlockSpec((2, 1, 128), lambda i: (0, 0, 0)),
                  pl.BlockSpec((2, 1, 128), lambda i: (0, 0, 0)),
                  pl.BlockSpec((ROW_BLK, IND), lambda i: (i, 0))],
        out_specs=pl.BlockSpec((ROW_BLK, IND), lambda i: (i, 0)),
        out_shape=jax.ShapeDtypeStruct((N, IND), f32),
    )(agg2, s1, s2, ghr, bhr, h)

    return (out, e)


# speedup vs baseline: 1.0403x; 1.0403x over previous
import jax
import jax.numpy as jnp
from jax import lax
from jax.experimental import pallas as pl
from jax.experimental.pallas import tpu as pltpu
from jax.experimental.pallas import tpu_sc as plsc

N = 10000
E = 160000
IND = 256
HID = 512
OUT = 64
H = 4
EPS = 1e-5

ROW_BLK = 5000
NB = N // ROW_BLK

CHUNK = 128
NSUB = 16
NCHUNK = -(-E // (CHUNK * NSUB)) * NSUB
E_PAD = NCHUNK * CHUNK
CPT = NCHUNK // NSUB
R_AGG = 10112


def _gram_affine_body(h_ref, w1_ref, g1_ref, b1_ref, a_ref, c_ref,
                      g_acc, s_acc):
    @pl.when(pl.program_id(0) == 0)
    def _():
        g_acc[...] = jnp.zeros_like(g_acc)
        s_acc[...] = jnp.zeros_like(s_acc)

    hb = h_ref[...]
    g_acc[...] += lax.dot_general(hb, hb, (((0,), (0,)), ((), ())),
                                  preferred_element_type=jnp.float32)
    s_acc[...] += jnp.sum(hb, axis=0, keepdims=True)

    @pl.when(pl.program_id(0) == NB - 1)
    def _():
        g = g_acc[...]
        s = s_acc[...]
        for i in range(H):
            w = w1_ref[i]
            gw = jnp.dot(g, w, preferred_element_type=jnp.float32)
            m2 = jnp.sum(w * gw, axis=0, keepdims=True) / N
            mu = jnp.dot(s, w, preferred_element_type=jnp.float32) / N
            rstd = lax.rsqrt(m2 - mu * mu + EPS)
            a = g1_ref[i] * rstd
            a_ref[i] = a
            c_ref[i] = b1_ref[i] - mu * a


def _mlp_body(h_ref, w1_ref, a_ref, c_ref, w2_ref, z_ref):
    hb = h_ref[...].astype(jnp.bfloat16)
    for i in range(H):
        t = jnp.dot(hb, w1_ref[i].astype(jnp.bfloat16),
                    preferred_element_type=jnp.float32)
        y = jnp.maximum(t * a_ref[i] + c_ref[i], 0.0)
        z_ref[:, i * OUT:(i + 1) * OUT] = jnp.dot(
            y.astype(jnp.bfloat16), w2_ref[i].astype(jnp.bfloat16),
            preferred_element_type=jnp.float32)


def _aggstat_body(a_ref, s1_ref, s2_ref):
    @pl.when(pl.program_id(0) == 0)
    def _():
        s1_ref[...] = jnp.zeros_like(s1_ref)
        s2_ref[...] = jnp.zeros_like(s2_ref)

    ab = a_ref[...]
    s1_ref[...] += jnp.sum(ab, axis=1, keepdims=True)
    s2_ref[...] += jnp.sum(ab * ab, axis=1, keepdims=True)


def _final_body(a_ref, s1_ref, s2_ref, g_ref, b_ref, h_ref, o_ref):
    mu = s1_ref[...] / N
    rstd = lax.rsqrt(s2_ref[...] / N - mu * mu + EPS)
    y = jnp.maximum((a_ref[...] - mu) * (rstd * g_ref[...]) + b_ref[...], 0.0)
    o_ref[...] = h_ref[...] + jnp.concatenate([y[0], y[1]], axis=1)


def _segsum_body(zt_hbm, idx_hbm, zeros_hbm, out_hbm,
                 idx_v, rows, agg_sh, gsem, ssem, isem):
    cid = lax.axis_index("c")
    sid = lax.axis_index("s")
    base = cid * NCHUNK + sid * CPT

    def start_idx(j, b):
        pltpu.async_copy(idx_hbm.at[base + j], idx_v[b], isem[b])

    def wait_idx(j, b):
        pltpu.make_async_copy(idx_hbm.at[base + j], idx_v[b], isem[b]).wait()

    def start_gather(b, r):
        pltpu.async_copy(zt_hbm.at[idx_v[b].at[0]], rows[r], gsem)

    def wait_gather(b, r):
        pltpu.make_async_copy(zt_hbm.at[idx_v[b].at[0]], rows[r],
                              gsem).wait()

    def start_scatter(b, r):
        pltpu.async_copy(rows[r], agg_sh.at[idx_v[b].at[1]], ssem, add=True)

    def wait_scatter(b, r):
        pltpu.make_async_copy(rows[r], agg_sh.at[idx_v[b].at[1]],
                              ssem).wait()

    zrows = R_AGG // NSUB
    for b in range(4):
        start_idx(b, b)
    pltpu.sync_copy(zeros_hbm.at[pl.ds(sid * zrows, zrows)],
                    agg_sh.at[pl.ds(sid * zrows, zrows)])
    wait_idx(0, 0)
    start_gather(0, 0)
    plsc.subcore_barrier()

    def half(j, b):
        r = b % 2
        wait_gather(b, r)

        @pl.when(j > 0)
        def _():
            wait_scatter((b + 3) % 4, 1 - r)

            @pl.when(j + 3 < CPT)
            def _():
                start_idx(j + 3, (b + 3) % 4)

        start_scatter(b, r)

        @pl.when(j + 1 < CPT)
        def _():
            wait_idx(j + 1, (b + 1) % 4)
            start_gather((b + 1) % 4, 1 - r)

    def body(k, carry):
        for b in range(4):
            j = 4 * k + b

            @pl.when(j < CPT)
            def _():
                half(j, b)

        return carry

    lax.fori_loop(0, (CPT + 3) // 4, body, 0)
    wait_scatter((CPT - 1) % 4, (CPT - 1) % 2)
    plsc.subcore_barrier()

    pltpu.sync_copy(agg_sh.at[pl.ds(sid * zrows, zrows)],
                    out_hbm.at[cid, pl.ds(sid * zrows, zrows)])


@jax.jit
def kernel(h, e, edge_index, W1, g1, b1, W2, gh, bh):
    f32 = jnp.float32

    a1, c1 = pl.pallas_call(
        _gram_affine_body,
        grid=(NB,),
        in_specs=[pl.BlockSpec((ROW_BLK, IND), lambda i: (i, 0)),
                  pl.BlockSpec((H, IND, HID), lambda i: (0, 0, 0)),
                  pl.BlockSpec((H, 1, HID), lambda i: (0, 0, 0)),
                  pl.BlockSpec((H, 1, HID), lambda i: (0, 0, 0))],
        out_specs=[pl.BlockSpec((H, 1, HID), lambda i: (0, 0, 0)),
                   pl.BlockSpec((H, 1, HID), lambda i: (0, 0, 0))],
        out_shape=[jax.ShapeDtypeStruct((H, 1, HID), f32),
                   jax.ShapeDtypeStruct((H, 1, HID), f32)],
        scratch_shapes=[pltpu.VMEM((IND, IND), f32),
                        pltpu.VMEM((1, IND), f32)],
    )(h, W1, g1.reshape(H, 1, HID), b1.reshape(H, 1, HID))

    z_all = pl.pallas_call(
        _mlp_body,
        grid=(NB,),
        in_specs=[pl.BlockSpec((ROW_BLK, IND), lambda i: (i, 0)),
                  pl.BlockSpec((H, IND, HID), lambda i: (0, 0, 0)),
                  pl.BlockSpec((H, 1, HID), lambda i: (0, 0, 0)),
                  pl.BlockSpec((H, 1, HID), lambda i: (0, 0, 0)),
                  pl.BlockSpec((H, HID, OUT), lambda i: (0, 0, 0))],
        out_specs=pl.BlockSpec((ROW_BLK, H * OUT), lambda i: (i, 0)),
        out_shape=jax.ShapeDtypeStruct((N, H * OUT), f32),
    )(h, W1, a1, c1, W2)

    src = edge_index[0]
    dst = edge_index[1]
    pad = E_PAD - E
    srcp = jnp.concatenate([src, jnp.zeros((pad,), jnp.int32)])
    dstp = jnp.concatenate(
        [dst, jnp.full((pad,), R_AGG - 1, jnp.int32)]).reshape(NCHUNK, CHUNK)
    g0 = (2 * srcp).reshape(NCHUNK, CHUNK)
    idx = jnp.stack([jnp.stack([g0, dstp], axis=1),
                     jnp.stack([g0 + 1, dstp], axis=1)])
    idx = idx.reshape(2 * NCHUNK, 2, CHUNK)
    zt = z_all.reshape(2 * N, 128)
    zeros = jnp.zeros((R_AGG, 128), f32)

    mesh = plsc.VectorSubcoreMesh(core_axis_name="c", subcore_axis_name="s")
    agg2 = pl.kernel(
        _segsum_body,
        out_type=jax.ShapeDtypeStruct((2, R_AGG, 128), f32),
        mesh=mesh,
        scratch_types=[
            [pltpu.VMEM((2, CHUNK), jnp.int32) for _ in range(4)],
            [pltpu.VMEM((CHUNK, 128), f32) for _ in range(2)],
            pltpu.VMEM_SHARED((R_AGG, 128), f32),
            pltpu.SemaphoreType.DMA,
            pltpu.SemaphoreType.DMA,
            [pltpu.SemaphoreType.DMA for _ in range(4)],
        ],
    )(zt, idx, zeros)

    s1, s2 = pl.pallas_call(
        _aggstat_body,
        grid=(NB,),
        in_specs=[pl.BlockSpec((2, ROW_BLK, 128), lambda i: (0, i, 0))],
        out_specs=[pl.BlockSpec((2, 1, 128), lambda i: (0, 0, 0)),
                   pl.BlockSpec((2, 1, 128), lambda i: (0, 0, 0))],
        out_shape=[jax.ShapeDtypeStruct((2, 1, 128), f32),
                   jax.ShapeDtypeStruct((2, 1, 128), f32)],
    )(agg2)

    ghr = gh.reshape(2, 1, 128)
    bhr = bh.reshape(2, 1, 128)
    out = pl.pallas_call(
        _final_body,
        grid=(NB,),
        in_specs=[pl.BlockSpec((2, ROW_BLK, 128), lambda i: (0, i, 0)),
                  pl.BlockSpec((2, 1, 128), lambda i: (0, 0, 0)),
                  pl.BlockSpec((2, 1, 128), lambda i: (0, 0, 0)),
                  pl.BlockSpec((2, 1, 128), lambda i: (0, 0, 0)),
                  pl.BlockSpec((2, 1, 128), lambda i: (0, 0, 0)),
                  pl.BlockSpec((ROW_BLK, IND), lambda i: (i, 0))],
        out_specs=pl.BlockSpec((ROW_BLK, IND), lambda i: (i, 0)),
        out_shape=jax.ShapeDtypeStruct((N, IND), f32),
    )(agg2, s1, s2, ghr, bhr, h)

    return (out, e)
